# Initial kernel scaffold; baseline (speedup 1.0000x reference)
import jax
import jax.numpy as jnp
from jax.experimental import pallas as pl


def kernel(x_user, x_item, edge_index_u2i, edge_index_i2u, W_in_user, b_in_user, W_in_item, b_in_item, l0_u2i_Wl, l0_u2i_Wr, l0_u2i_b, l0_i2u_Wl, l0_i2u_Wr, l0_i2u_b, l1_u2i_Wl, l1_u2i_Wr, l1_u2i_b, l1_i2u_Wl, l1_i2u_Wr, l1_i2u_b, W_out, b_out):
    raise NotImplementedError("write your pallas kernel here")


# speedup vs baseline: 6.3227x; 6.3227x over previous
import functools

import jax
import jax.numpy as jnp
from jax import lax
from jax.experimental import pallas as pl
from jax.experimental.pallas import tpu as pltpu
from jax.experimental.pallas import tpu_sc as plsc

N_NODE = 10000
N_ACC = 10112
E = 320000
CHUNK = 128
NC, NS = 2, 16
NW = NC * NS
E_PAD = 327680
CPT = E_PAD // (NW * CHUNK)
ZR = N_ACC // NS

_MESH = plsc.VectorSubcoreMesh(core_axis_name="c", subcore_axis_name="s")

_ZBLOCKS = [(64, 9), (56, 1)]

CH = 128
CPT2 = E_PAD // (NW * CH)
NP = CPT2 // 2
TW = 80


def _pad_edges(ei):
    src = ei[0].astype(jnp.int32)
    dst = ei[1].astype(jnp.int32)
    pad = E_PAD - E
    src = jnp.concatenate([src, jnp.zeros((pad,), jnp.int32)])
    dst = jnp.concatenate([dst, jnp.full((pad,), N_NODE, jnp.int32)])
    return src, dst


def _staged_copy(src_ref, dst_ref, stage, src0, dst0):
    off = 0
    for rows, cnt in _ZBLOCKS:
        for _ in range(cnt):
            pltpu.sync_copy(src_ref.at[pl.ds(src0 + off, rows)],
                            stage.at[pl.ds(0, rows)])
            pltpu.sync_copy(stage.at[pl.ds(0, rows)],
                            dst_ref.at[pl.ds(dst0 + off, rows)])
            off += rows


def _seg_body(tbl, src_h, dst_h, z80,
              out_s,
              isA, idA, isB, idB, rows0, rows1, acc,
              gsem0, gsem1, isem0, isem1):
    c = lax.axis_index("c")
    s = lax.axis_index("s")
    row0 = s * ZR

    pltpu.sync_copy(z80.at[pl.ds(0, CH)], rows0)
    off = 0
    for rows, cnt in _ZBLOCKS:
        for _ in range(cnt):
            pltpu.sync_copy(rows0.at[pl.ds(0, rows)],
                            acc.at[pl.ds(row0 + off, rows)])
            off += rows
    plsc.subcore_barrier()

    wid = s * NC + c
    e0 = wid * (CPT2 * CH)

    def idx_start(k, is_, id_, sem):
        e = e0 + k * CH
        pltpu.async_copy(src_h.at[pl.ds(e, CH)], is_, sem)
        pltpu.async_copy(dst_h.at[pl.ds(e, CH)], id_, sem)

    def idx_wait(is_, id_, sem):
        pltpu.make_async_copy(src_h.at[pl.ds(0, CH)], is_, sem).wait()
        pltpu.make_async_copy(src_h.at[pl.ds(0, CH)], id_, sem).wait()

    def rows_wait(buf, sem):
        pltpu.make_async_copy(z80.at[pl.ds(0, CH)], buf, sem).wait()

    idx_start(0, isA, idA, isem0)
    idx_wait(isA, idA, isem0)
    pltpu.async_copy(tbl.at[isA], rows0, gsem0)
    idx_start(1, isB, idB, isem1)

    @pl.loop(0, NP - 1)
    def _(p):
        rows_wait(rows0, gsem0)
        idx_wait(isB, idB, isem1)
        pltpu.async_copy(tbl.at[isB], rows1, gsem1)
        pltpu.sync_copy(rows0, acc.at[idA], add=True)
        idx_start(2 * p + 2, isA, idA, isem0)
        rows_wait(rows1, gsem1)
        idx_wait(isA, idA, isem0)
        pltpu.async_copy(tbl.at[isA], rows0, gsem0)
        pltpu.sync_copy(rows1, acc.at[idB], add=True)
        idx_start(2 * p + 3, isB, idB, isem1)

    rows_wait(rows0, gsem0)
    idx_wait(isB, idB, isem1)
    pltpu.async_copy(tbl.at[isB], rows1, gsem1)
    pltpu.sync_copy(rows0, acc.at[idA], add=True)
    rows_wait(rows1, gsem1)
    pltpu.sync_copy(rows1, acc.at[idB], add=True)

    plsc.subcore_barrier()
    _staged_copy(acc, out_s, rows0, row0, c * N_ACC + row0)


_seg_sum = pl.kernel(
    _seg_body,
    out_type=jax.ShapeDtypeStruct((NC * N_ACC, TW), jnp.float32),
    mesh=_MESH,
    compiler_params=pltpu.CompilerParams(use_tc_tiling_on_sc=False),
    scratch_types=[
        pltpu.VMEM((CH,), jnp.int32),
        pltpu.VMEM((CH,), jnp.int32),
        pltpu.VMEM((CH,), jnp.int32),
        pltpu.VMEM((CH,), jnp.int32),
        pltpu.VMEM((CH, TW), jnp.float32),
        pltpu.VMEM((CH, TW), jnp.float32),
        pltpu.VMEM_SHARED((N_ACC, TW), jnp.float32),
        pltpu.SemaphoreType.DMA,
        pltpu.SemaphoreType.DMA,
        pltpu.SemaphoreType.DMA,
        pltpu.SemaphoreType.DMA,
    ],
)


_BR = 128
_NB = N_ACC // _BR


def _table_body(x_ref, w_ref, b_ref, wl_ref, w1_ref, wo_ref, t_ref):
    y = jnp.maximum(
        jnp.dot(x_ref[...], w_ref[...], preferred_element_type=jnp.float32)
        + b_ref[...], 0.0)
    a = jnp.dot(wl_ref[...],
                jnp.dot(w1_ref[...], wo_ref[...],
                        preferred_element_type=jnp.float32),
                preferred_element_type=jnp.float32)
    t = jnp.dot(y, a, preferred_element_type=jnp.float32)
    t_ref[...] = jnp.concatenate(
        [t, jnp.ones((t.shape[0], 1), jnp.float32),
         jnp.zeros((t.shape[0], TW - 65), jnp.float32)], axis=1)


def _table(x, w, b, wl, w1, wo):
    return pl.pallas_call(
        _table_body,
        grid=(N_NODE // 2000,),
        in_specs=[
            pl.BlockSpec((2000, 128), lambda i: (i, 0)),
            pl.BlockSpec((128, 128), lambda i: (0, 0)),
            pl.BlockSpec((1, 128), lambda i: (0, 0)),
            pl.BlockSpec((128, 128), lambda i: (0, 0)),
            pl.BlockSpec((128, 128), lambda i: (0, 0)),
            pl.BlockSpec((128, 64), lambda i: (0, 0)),
        ],
        out_specs=pl.BlockSpec((2000, TW), lambda i: (i, 0)),
        out_shape=jax.ShapeDtypeStruct((N_NODE, TW), jnp.float32),
    )(x, w, b.reshape(1, 128), wl, w1, wo)


def _mix_body(sa_ref, sb_ref, x_ref, win_ref, bin_ref, wr_ref, b_ref,
              w1_ref, wl2_ref, w12_ref, wo_ref, o_ref):
    wc = jnp.dot(w1_ref[...], wo_ref[...], preferred_element_type=jnp.float32)
    wrc = jnp.dot(w12_ref[...], wo_ref[...], preferred_element_type=jnp.float32)
    wcomb = (jnp.dot(wr_ref[...], wc, preferred_element_type=jnp.float32)
             + jnp.dot(wl2_ref[...], wrc, preferred_element_type=jnp.float32))
    bc = jnp.dot(b_ref[...], wc, preferred_element_type=jnp.float32)
    y = jnp.maximum(
        jnp.dot(x_ref[...], win_ref[...], preferred_element_type=jnp.float32)
        + bin_ref[...], 0.0)
    ssum = sa_ref[...] + sb_ref[...]
    deg = jnp.maximum(ssum[:, 64:65], 1.0)
    r = (ssum[:, :64] / deg
         + jnp.dot(y, wcomb, preferred_element_type=jnp.float32)
         + bc)
    o_ref[...] = jnp.concatenate(
        [r, jnp.ones((r.shape[0], 1), jnp.float32),
         jnp.zeros((r.shape[0], TW - 65), jnp.float32)], axis=1)


def _mix(seg, x, win, bin_, wr, b, w1, wl2, w12, wo):
    return pl.pallas_call(
        _mix_body,
        grid=(_NB,),
        in_specs=[
            pl.BlockSpec((_BR, TW), lambda i: (i, 0)),
            pl.BlockSpec((_BR, TW), lambda i: (i + _NB, 0)),
            pl.BlockSpec((_BR, 128), lambda i: (i, 0)),
            pl.BlockSpec((128, 128), lambda i: (0, 0)),
            pl.BlockSpec((1, 128), lambda i: (0, 0)),
            pl.BlockSpec((128, 128), lambda i: (0, 0)),
            pl.BlockSpec((1, 128), lambda i: (0, 0)),
            pl.BlockSpec((128, 128), lambda i: (0, 0)),
            pl.BlockSpec((128, 128), lambda i: (0, 0)),
            pl.BlockSpec((128, 128), lambda i: (0, 0)),
            pl.BlockSpec((128, 64), lambda i: (0, 0)),
        ],
        out_specs=pl.BlockSpec((_BR, TW), lambda i: (i, 0)),
        out_shape=jax.ShapeDtypeStruct((N_NODE, TW), jnp.float32),
    )(seg, seg, x, win, bin_.reshape(1, 128), wr, b.reshape(1, 128), w1,
      wl2, w12, wo)


def _final_body(sa_ref, sb_ref, x_ref, win_ref, bin_ref, wr2_ref, b2_ref,
                w12_ref, b1_ref, wo_ref, bo_ref, o_ref):
    wrc = jnp.dot(w12_ref[...], wo_ref[...], preferred_element_type=jnp.float32)
    bias = (jnp.dot(b2_ref[...], wrc, preferred_element_type=jnp.float32)
            + jnp.dot(b1_ref[...], wo_ref[...],
                      preferred_element_type=jnp.float32)
            + bo_ref[...])
    y = jnp.maximum(
        jnp.dot(x_ref[...], win_ref[...], preferred_element_type=jnp.float32)
        + bin_ref[...], 0.0)
    ssum = sa_ref[...] + sb_ref[...]
    deg = jnp.maximum(ssum[:, 64:65], 1.0)
    o_ref[...] = (ssum[:, :64] / deg
                  + jnp.dot(y, jnp.dot(
                      wr2_ref[...], wrc, preferred_element_type=jnp.float32),
                      preferred_element_type=jnp.float32)
                  + bias)


def _final(seg, x, win, bin_, wr2, b2, w12, b1, wo, bo):
    return pl.pallas_call(
        _final_body,
        grid=(_NB,),
        in_specs=[
            pl.BlockSpec((_BR, TW), lambda i: (i, 0)),
            pl.BlockSpec((_BR, TW), lambda i: (i + _NB, 0)),
            pl.BlockSpec((_BR, 128), lambda i: (i, 0)),
            pl.BlockSpec((128, 128), lambda i: (0, 0)),
            pl.BlockSpec((1, 128), lambda i: (0, 0)),
            pl.BlockSpec((128, 128), lambda i: (0, 0)),
            pl.BlockSpec((1, 128), lambda i: (0, 0)),
            pl.BlockSpec((128, 128), lambda i: (0, 0)),
            pl.BlockSpec((1, 128), lambda i: (0, 0)),
            pl.BlockSpec((128, 64), lambda i: (0, 0)),
            pl.BlockSpec((1, 64), lambda i: (0, 0)),
        ],
        out_specs=pl.BlockSpec((_BR, 64), lambda i: (i, 0)),
        out_shape=jax.ShapeDtypeStruct((N_NODE, 64), jnp.float32),
    )(seg, seg, x, win, bin_.reshape(1, 128), wr2, b2.reshape(1, 128), w12,
      b1.reshape(1, 128), wo, bo.reshape(1, 64))


def kernel(x_user, x_item, edge_index_u2i, edge_index_i2u,
           W_in_user, b_in_user, W_in_item, b_in_item,
           l0_u2i_Wl, l0_u2i_Wr, l0_u2i_b, l0_i2u_Wl, l0_i2u_Wr, l0_i2u_b,
           l1_u2i_Wl, l1_u2i_Wr, l1_u2i_b, l1_i2u_Wl, l1_i2u_Wr, l1_i2u_b,
           W_out, b_out):
    su2i, du2i = _pad_edges(edge_index_u2i)
    si2u, di2u = _pad_edges(edge_index_i2u)
    z80 = jnp.zeros((N_ACC, TW), jnp.float32)

    t_u = _table(x_user, W_in_user, b_in_user,
                 l0_u2i_Wl, l1_i2u_Wl, W_out)

    s_i = _seg_sum(t_u, su2i, du2i, z80)
    comb = _mix(s_i, x_item, W_in_item, b_in_item, l0_u2i_Wr, l0_u2i_b,
                l1_i2u_Wl, l0_i2u_Wl, l1_i2u_Wr, W_out)
    s_c = _seg_sum(comb, si2u, di2u, z80)

    return _final(s_c, x_user, W_in_user, b_in_user, l0_i2u_Wr, l0_i2u_b,
                  l1_i2u_Wr, l1_i2u_b, W_out, b_out)
